# baseline (speedup 1.0000x reference)
import functools

import jax
import jax.numpy as jnp
from jax import lax
from jax.experimental import pallas as pl
from jax.experimental.pallas import tpu as pltpu
from jax.experimental.pallas import tpu_sc as plsc

N = 10000
E = 320000
NFEAT = 128
NHID = 128
NCLASS = 64

NC = 2
NS = 16
NW = NC * NS
CHUNK = 128
K = 80
KF = 100
KS = 60
FAST_C = 0
TOT_CHUNKS = NW * K
E_PAD = TOT_CHUNKS * CHUNK
NACC = 10240
ROWS_PER_TILE = NACC // NS
COPIES_PER_TILE = ROWS_PER_TILE // CHUNK


def _make_spmm(D):
  mesh = plsc.VectorSubcoreMesh(core_axis_name="c", subcore_axis_name="s")

  @functools.partial(
      pl.kernel,
      out_type=jax.ShapeDtypeStruct((NC, NACC, D), jnp.float32),
      mesh=mesh,
      compiler_params=pltpu.CompilerParams(use_tc_tiling_on_sc=False),
      scratch_types=[
          pltpu.VMEM((KF, CHUNK), jnp.int32),
          pltpu.VMEM((KF, CHUNK), jnp.int32),
          pltpu.VMEM((CHUNK, D), jnp.float32),
          pltpu.VMEM_SHARED((NACC, D), jnp.float32),
          pltpu.SemaphoreType.DMA,
      ],
  )
  def spmm(support_hbm, src_hbm, dst_hbm, out_hbm, src_v, dst_v, rows_v,
           acc, sem):
    c = lax.axis_index("c")
    s = lax.axis_index("s")

    zeros16 = jnp.zeros((16,), jnp.float32)

    def zero_row(i, carry):
      for t in range(D // 16):
        rows_v[i, pl.ds(t * 16, 16)] = zeros16
      return carry

    lax.fori_loop(0, CHUNK, zero_row, 0)
    base = s * ROWS_PER_TILE
    for t in range(COPIES_PER_TILE):
      pltpu.sync_copy(rows_v, acc.at[pl.ds(base + t * CHUNK, CHUNK)])
    plsc.subcore_barrier()

    def run(kc, chunk0):
      pltpu.sync_copy(src_hbm.at[pl.ds(chunk0, kc)], src_v.at[pl.ds(0, kc)])
      pltpu.sync_copy(dst_hbm.at[pl.ds(chunk0, kc)], dst_v.at[pl.ds(0, kc)])

      def body(j, carry):
        pltpu.async_copy(support_hbm.at[src_v.at[j]], rows_v, sem).wait()
        pltpu.sync_copy(rows_v, acc.at[dst_v.at[j]], add=True)
        return carry

      lax.fori_loop(0, kc, body, 0)

    @pl.when(c == FAST_C)
    def _():
      run(KF, s * KF)

    @pl.when(c != FAST_C)
    def _():
      run(KS, NS * KF + s * KS)

    plsc.subcore_barrier()

    for t in range(COPIES_PER_TILE):
      r0 = base + t * CHUNK
      pltpu.sync_copy(acc.at[pl.ds(r0, CHUNK)], rows_v)
      pltpu.sync_copy(rows_v, out_hbm.at[c, pl.ds(r0, CHUNK)])

  return spmm


_spmm128 = _make_spmm(NHID)
_spmm64 = _make_spmm(NCLASS)

_ROWS_BLK = 1000
_GRID = N // _ROWS_BLK


def _mm_first(x, W):
  def body(x_ref, w_ref, o_ref):
    o_ref[...] = jnp.dot(x_ref[...], w_ref[...],
                         preferred_element_type=jnp.float32)

  return pl.pallas_call(
      body,
      grid=(_GRID,),
      in_specs=[
          pl.BlockSpec((_ROWS_BLK, x.shape[1]), lambda i: (i, 0)),
          pl.BlockSpec(W.shape, lambda i: (0, 0)),
      ],
      out_specs=pl.BlockSpec((_ROWS_BLK, W.shape[1]), lambda i: (i, 0)),
      out_shape=jax.ShapeDtypeStruct((N, W.shape[1]), jnp.float32),
  )(x, W)


def _mm_agg(agg, b, W):
  D = agg.shape[2]

  def body(a_ref, b_ref, w_ref, o_ref):
    h = a_ref[0] + a_ref[1] + b_ref[...]
    o_ref[...] = jnp.dot(h, w_ref[...], preferred_element_type=jnp.float32)

  return pl.pallas_call(
      body,
      grid=(_GRID,),
      in_specs=[
          pl.BlockSpec((NC, _ROWS_BLK, D), lambda i: (0, i, 0)),
          pl.BlockSpec((1, D), lambda i: (0, 0)),
          pl.BlockSpec(W.shape, lambda i: (0, 0)),
      ],
      out_specs=pl.BlockSpec((_ROWS_BLK, W.shape[1]), lambda i: (i, 0)),
      out_shape=jax.ShapeDtypeStruct((N, W.shape[1]), jnp.float32),
  )(agg, b.reshape(1, D), W)


def _softmax_out(agg, b):
  D = agg.shape[2]

  def body(a_ref, b_ref, o_ref):
    z = a_ref[0] + a_ref[1] + b_ref[...]
    z = z - jnp.max(z, axis=1, keepdims=True)
    e = jnp.exp(z)
    o_ref[...] = e / jnp.sum(e, axis=1, keepdims=True)

  return pl.pallas_call(
      body,
      grid=(_GRID,),
      in_specs=[
          pl.BlockSpec((NC, _ROWS_BLK, D), lambda i: (0, i, 0)),
          pl.BlockSpec((1, D), lambda i: (0, 0)),
      ],
      out_specs=pl.BlockSpec((_ROWS_BLK, D), lambda i: (i, 0)),
      out_shape=jax.ShapeDtypeStruct((N, D), jnp.float32),
  )(agg, b.reshape(1, D))


def kernel(x, edge_index, W1, b1, W2, b2, W3, b3):
  src = edge_index[0]
  dst = edge_index[1]
  pad = E_PAD - E
  src_p = jnp.concatenate([src, jnp.zeros((pad,), jnp.int32)])
  dst_p = jnp.concatenate([dst, jnp.full((pad,), N, jnp.int32)])
  src_p = src_p.reshape(TOT_CHUNKS, CHUNK)
  dst_p = dst_p.reshape(TOT_CHUNKS, CHUNK)

  support1 = _mm_first(x, W1)
  agg1 = _spmm128(support1, src_p, dst_p)
  support2 = _mm_agg(agg1, b1, W2)
  agg2 = _spmm128(support2, src_p, dst_p)
  support3 = _mm_agg(agg2, b2, W3)
  agg3 = _spmm64(support3, src_p, dst_p)
  return _softmax_out(agg3, b3)


# speedup vs baseline: 1.0215x; 1.0215x over previous
import functools

import jax
import jax.numpy as jnp
from jax import lax
from jax.experimental import pallas as pl
from jax.experimental.pallas import tpu as pltpu
from jax.experimental.pallas import tpu_sc as plsc

N = 10000
E = 320000
NFEAT = 128
NHID = 128
NCLASS = 64

NC = 2
NS = 16
NW = NC * NS
CHUNK = 128
K = 80
KF = 100
KS = 60
FAST_C = 1
TOT_CHUNKS = NW * K
E_PAD = TOT_CHUNKS * CHUNK
NACC = 10240
ROWS_PER_TILE = NACC // NS
COPIES_PER_TILE = ROWS_PER_TILE // CHUNK


def _make_spmm(D):
  mesh = plsc.VectorSubcoreMesh(core_axis_name="c", subcore_axis_name="s")

  @functools.partial(
      pl.kernel,
      out_type=jax.ShapeDtypeStruct((NC, NACC, D), jnp.float32),
      mesh=mesh,
      compiler_params=pltpu.CompilerParams(use_tc_tiling_on_sc=False),
      scratch_types=[
          pltpu.VMEM((KF, CHUNK), jnp.int32),
          pltpu.VMEM((KF, CHUNK), jnp.int32),
          pltpu.VMEM((CHUNK, D), jnp.float32),
          pltpu.VMEM_SHARED((NACC, D), jnp.float32),
          pltpu.SemaphoreType.DMA,
      ],
  )
  def spmm(support_hbm, src_hbm, dst_hbm, out_hbm, src_v, dst_v, rows_v,
           acc, sem):
    c = lax.axis_index("c")
    s = lax.axis_index("s")

    zeros16 = jnp.zeros((16,), jnp.float32)

    def zero_row(i, carry):
      for t in range(D // 16):
        rows_v[i, pl.ds(t * 16, 16)] = zeros16
      return carry

    lax.fori_loop(0, CHUNK, zero_row, 0)
    base = s * ROWS_PER_TILE
    for t in range(COPIES_PER_TILE):
      pltpu.sync_copy(rows_v, acc.at[pl.ds(base + t * CHUNK, CHUNK)])
    plsc.subcore_barrier()

    def run(kc, chunk0):
      pltpu.sync_copy(src_hbm.at[pl.ds(chunk0, kc)], src_v.at[pl.ds(0, kc)])
      pltpu.sync_copy(dst_hbm.at[pl.ds(chunk0, kc)], dst_v.at[pl.ds(0, kc)])

      def body(j, carry):
        pltpu.async_copy(support_hbm.at[src_v.at[j]], rows_v, sem).wait()
        pltpu.sync_copy(rows_v, acc.at[dst_v.at[j]], add=True)
        return carry

      lax.fori_loop(0, kc, body, 0)

    @pl.when(c == FAST_C)
    def _():
      run(KF, s * KF)

    @pl.when(c != FAST_C)
    def _():
      run(KS, NS * KF + s * KS)

    plsc.subcore_barrier()

    for t in range(COPIES_PER_TILE):
      r0 = base + t * CHUNK
      pltpu.sync_copy(acc.at[pl.ds(r0, CHUNK)], rows_v)
      pltpu.sync_copy(rows_v, out_hbm.at[c, pl.ds(r0, CHUNK)])

  return spmm


_spmm128 = _make_spmm(NHID)
_spmm64 = _make_spmm(NCLASS)

_ROWS_BLK = 1000
_GRID = N // _ROWS_BLK


def _mm_first(x, W):
  def body(x_ref, w_ref, o_ref):
    o_ref[...] = jnp.dot(x_ref[...], w_ref[...],
                         preferred_element_type=jnp.float32)

  return pl.pallas_call(
      body,
      grid=(_GRID,),
      in_specs=[
          pl.BlockSpec((_ROWS_BLK, x.shape[1]), lambda i: (i, 0)),
          pl.BlockSpec(W.shape, lambda i: (0, 0)),
      ],
      out_specs=pl.BlockSpec((_ROWS_BLK, W.shape[1]), lambda i: (i, 0)),
      out_shape=jax.ShapeDtypeStruct((N, W.shape[1]), jnp.float32),
  )(x, W)


def _mm_agg(agg, b, W):
  D = agg.shape[2]

  def body(a_ref, b_ref, w_ref, o_ref):
    h = a_ref[0] + a_ref[1] + b_ref[...]
    o_ref[...] = jnp.dot(h, w_ref[...], preferred_element_type=jnp.float32)

  return pl.pallas_call(
      body,
      grid=(_GRID,),
      in_specs=[
          pl.BlockSpec((NC, _ROWS_BLK, D), lambda i: (0, i, 0)),
          pl.BlockSpec((1, D), lambda i: (0, 0)),
          pl.BlockSpec(W.shape, lambda i: (0, 0)),
      ],
      out_specs=pl.BlockSpec((_ROWS_BLK, W.shape[1]), lambda i: (i, 0)),
      out_shape=jax.ShapeDtypeStruct((N, W.shape[1]), jnp.float32),
  )(agg, b.reshape(1, D), W)


def _softmax_out(agg, b):
  D = agg.shape[2]

  def body(a_ref, b_ref, o_ref):
    z = a_ref[0] + a_ref[1] + b_ref[...]
    z = z - jnp.max(z, axis=1, keepdims=True)
    e = jnp.exp(z)
    o_ref[...] = e / jnp.sum(e, axis=1, keepdims=True)

  return pl.pallas_call(
      body,
      grid=(_GRID,),
      in_specs=[
          pl.BlockSpec((NC, _ROWS_BLK, D), lambda i: (0, i, 0)),
          pl.BlockSpec((1, D), lambda i: (0, 0)),
      ],
      out_specs=pl.BlockSpec((_ROWS_BLK, D), lambda i: (i, 0)),
      out_shape=jax.ShapeDtypeStruct((N, D), jnp.float32),
  )(agg, b.reshape(1, D))


def kernel(x, edge_index, W1, b1, W2, b2, W3, b3):
  src = edge_index[0]
  dst = edge_index[1]
  pad = E_PAD - E
  src_p = jnp.concatenate([src, jnp.zeros((pad,), jnp.int32)])
  dst_p = jnp.concatenate([dst, jnp.full((pad,), N, jnp.int32)])
  src_p = src_p.reshape(TOT_CHUNKS, CHUNK)
  dst_p = dst_p.reshape(TOT_CHUNKS, CHUNK)

  support1 = _mm_first(x, W1)
  agg1 = _spmm128(support1, src_p, dst_p)
  support2 = _mm_agg(agg1, b1, W2)
  agg2 = _spmm128(support2, src_p, dst_p)
  support3 = _mm_agg(agg2, b2, W3)
  agg3 = _spmm64(support3, src_p, dst_p)
  return _softmax_out(agg3, b3)


# speedup vs baseline: 2.1384x; 2.0934x over previous
import functools

import jax
import jax.numpy as jnp
from jax import lax
from jax.experimental import pallas as pl
from jax.experimental.pallas import tpu as pltpu
from jax.experimental.pallas import tpu_sc as plsc

N = 10000
E = 320000
NFEAT = 128
NHID = 128
NCLASS = 64

NC = 2
NS = 16
NW = NC * NS
CHUNK = 128
K = 80
TOT_CHUNKS = NW * K
E_PAD = TOT_CHUNKS * CHUNK
NACC = 10240
ROWS_PER_TILE = NACC // NS
COPIES_PER_TILE = ROWS_PER_TILE // CHUNK


def _make_spmm(D):
  mesh = plsc.VectorSubcoreMesh(core_axis_name="c", subcore_axis_name="s")

  @functools.partial(
      pl.kernel,
      out_type=jax.ShapeDtypeStruct((NC, NACC, D), jnp.float32),
      mesh=mesh,
      compiler_params=pltpu.CompilerParams(use_tc_tiling_on_sc=False),
      scratch_types=[
          pltpu.VMEM((K, CHUNK), jnp.int32),
          pltpu.VMEM((K, CHUNK), jnp.int32),
          pltpu.VMEM((CHUNK, D), jnp.float32),
          pltpu.VMEM_SHARED((NACC, D), jnp.float32),
          pltpu.SemaphoreType.DMA,
      ],
  )
  def spmm(support_hbm, src_hbm, dst_hbm, out_hbm, src_v, dst_v, rows_v,
           acc, sem):
    c = lax.axis_index("c")
    s = lax.axis_index("s")

    zeros16 = jnp.zeros((16,), jnp.float32)

    def zero_row(i, carry):
      for t in range(D // 16):
        rows_v[i, pl.ds(t * 16, 16)] = zeros16
      return carry

    lax.fori_loop(0, CHUNK, zero_row, 0)
    base = s * ROWS_PER_TILE
    for t in range(COPIES_PER_TILE):
      pltpu.sync_copy(rows_v, acc.at[pl.ds(base + t * CHUNK, CHUNK)])
    plsc.subcore_barrier()

    chunk0 = (c * NS + s) * K
    pltpu.sync_copy(src_hbm.at[pl.ds(chunk0, K)], src_v)
    pltpu.sync_copy(dst_hbm.at[pl.ds(chunk0, K)], dst_v)

    def body(j, carry):
      pltpu.async_copy(support_hbm.at[src_v.at[j]], rows_v, sem).wait()
      pltpu.sync_copy(rows_v, acc.at[dst_v.at[j]], add=True)
      return carry

    lax.fori_loop(0, K, body, 0)
    plsc.subcore_barrier()

    for t in range(COPIES_PER_TILE):
      r0 = base + t * CHUNK
      pltpu.sync_copy(acc.at[pl.ds(r0, CHUNK)], rows_v)
      pltpu.sync_copy(rows_v, out_hbm.at[c, pl.ds(r0, CHUNK)])

  return spmm


_spmm128 = _make_spmm(NHID)
_spmm64 = _make_spmm(NCLASS)

_ROWS_BLK = 1000
_GRID = N // _ROWS_BLK


def _mm_first(x, W):
  def body(x_ref, w_ref, o_ref):
    o_ref[...] = jnp.dot(x_ref[...], w_ref[...],
                         preferred_element_type=jnp.float32)

  return pl.pallas_call(
      body,
      grid=(_GRID,),
      in_specs=[
          pl.BlockSpec((_ROWS_BLK, x.shape[1]), lambda i: (i, 0)),
          pl.BlockSpec(W.shape, lambda i: (0, 0)),
      ],
      out_specs=pl.BlockSpec((_ROWS_BLK, W.shape[1]), lambda i: (i, 0)),
      out_shape=jax.ShapeDtypeStruct((N, W.shape[1]), jnp.float32),
  )(x, W)


def _mm_agg(agg, b, W):
  D = agg.shape[2]

  def body(a_ref, b_ref, w_ref, o_ref):
    h = a_ref[0] + a_ref[1] + b_ref[...]
    o_ref[...] = jnp.dot(h, w_ref[...], preferred_element_type=jnp.float32)

  return pl.pallas_call(
      body,
      grid=(_GRID,),
      in_specs=[
          pl.BlockSpec((NC, _ROWS_BLK, D), lambda i: (0, i, 0)),
          pl.BlockSpec((1, D), lambda i: (0, 0)),
          pl.BlockSpec(W.shape, lambda i: (0, 0)),
      ],
      out_specs=pl.BlockSpec((_ROWS_BLK, W.shape[1]), lambda i: (i, 0)),
      out_shape=jax.ShapeDtypeStruct((N, W.shape[1]), jnp.float32),
  )(agg, b.reshape(1, D), W)


def _softmax_out(agg, b):
  D = agg.shape[2]

  def body(a_ref, b_ref, o_ref):
    z = a_ref[0] + a_ref[1] + b_ref[...]
    z = z - jnp.max(z, axis=1, keepdims=True)
    e = jnp.exp(z)
    o_ref[...] = e / jnp.sum(e, axis=1, keepdims=True)

  return pl.pallas_call(
      body,
      grid=(_GRID,),
      in_specs=[
          pl.BlockSpec((NC, _ROWS_BLK, D), lambda i: (0, i, 0)),
          pl.BlockSpec((1, D), lambda i: (0, 0)),
      ],
      out_specs=pl.BlockSpec((_ROWS_BLK, D), lambda i: (i, 0)),
      out_shape=jax.ShapeDtypeStruct((N, D), jnp.float32),
  )(agg, b.reshape(1, D))


def kernel(x, edge_index, W1, b1, W2, b2, W3, b3):
  src = edge_index[0]
  dst = edge_index[1]
  pad = E_PAD - E
  r = jnp.arange(pad, dtype=jnp.int32)
  src_p = jnp.concatenate([src, r % N])
  dst_p = jnp.concatenate([dst, N + (r % (NACC - N))])
  src_p = src_p.reshape(TOT_CHUNKS, CHUNK)
  dst_p = dst_p.reshape(TOT_CHUNKS, CHUNK)

  support1 = _mm_first(x, W1)
  agg1 = _spmm128(support1, src_p, dst_p)
  support2 = _mm_agg(agg1, b1, W2)
  agg2 = _spmm128(support2, src_p, dst_p)
  support3 = _mm_agg(agg2, b2, W3)
  agg3 = _spmm64(support3, src_p, dst_p)
  return _softmax_out(agg3, b3)


# speedup vs baseline: 2.1515x; 1.0061x over previous
import functools

import jax
import jax.numpy as jnp
from jax import lax
from jax.experimental import pallas as pl
from jax.experimental.pallas import tpu as pltpu
from jax.experimental.pallas import tpu_sc as plsc

N = 10000
E = 320000
NFEAT = 128
NHID = 128
NCLASS = 64

NC = 2
NS = 16
NW = NC * NS
CHUNK = 128
TOT_CHUNKS = E // CHUNK
KB = TOT_CHUNKS // NW
NA = TOT_CHUNKS - NW * KB
KA = KB + 1
NACC = 10240
ROWS_PER_TILE = NACC // NS
COPIES_PER_TILE = ROWS_PER_TILE // CHUNK


def _make_spmm(D):
  mesh = plsc.VectorSubcoreMesh(core_axis_name="c", subcore_axis_name="s")

  @functools.partial(
      pl.kernel,
      out_type=jax.ShapeDtypeStruct((NC, NACC, D), jnp.float32),
      mesh=mesh,
      compiler_params=pltpu.CompilerParams(use_tc_tiling_on_sc=False),
      scratch_types=[
          pltpu.VMEM((KA, CHUNK), jnp.int32),
          pltpu.VMEM((KA, CHUNK), jnp.int32),
          pltpu.VMEM((CHUNK, D), jnp.float32),
          pltpu.VMEM_SHARED((NACC, D), jnp.float32),
          pltpu.SemaphoreType.DMA,
      ],
  )
  def spmm(support_hbm, src_hbm, dst_hbm, out_hbm, src_v, dst_v, rows_v,
           acc, sem):
    c = lax.axis_index("c")
    s = lax.axis_index("s")

    zeros16 = jnp.zeros((16,), jnp.float32)

    def zero_row(i, carry):
      for t in range(D // 16):
        rows_v[i, pl.ds(t * 16, 16)] = zeros16
      return carry

    lax.fori_loop(0, CHUNK, zero_row, 0)
    base = s * ROWS_PER_TILE
    for t in range(COPIES_PER_TILE):
      pltpu.sync_copy(rows_v, acc.at[pl.ds(base + t * CHUNK, CHUNK)])
    plsc.subcore_barrier()

    w = c * NS + s
    chunk0 = w * KB + jnp.minimum(w, NA)

    def run(kc):
      pltpu.sync_copy(src_hbm.at[pl.ds(chunk0, kc)], src_v.at[pl.ds(0, kc)])
      pltpu.sync_copy(dst_hbm.at[pl.ds(chunk0, kc)], dst_v.at[pl.ds(0, kc)])

      def body(j, carry):
        pltpu.async_copy(support_hbm.at[src_v.at[j]], rows_v, sem).wait()
        pltpu.sync_copy(rows_v, acc.at[dst_v.at[j]], add=True)
        return carry

      lax.fori_loop(0, kc, body, 0)

    @pl.when(w < NA)
    def _():
      run(KA)

    @pl.when(w >= NA)
    def _():
      run(KB)

    plsc.subcore_barrier()

    for t in range(COPIES_PER_TILE):
      r0 = base + t * CHUNK
      pltpu.sync_copy(acc.at[pl.ds(r0, CHUNK)], rows_v)
      pltpu.sync_copy(rows_v, out_hbm.at[c, pl.ds(r0, CHUNK)])

  return spmm


_spmm128 = _make_spmm(NHID)
_spmm64 = _make_spmm(NCLASS)

_ROWS_BLK = 1000
_GRID = N // _ROWS_BLK


def _mm_first(x, W):
  def body(x_ref, w_ref, o_ref):
    o_ref[...] = jnp.dot(x_ref[...], w_ref[...],
                         preferred_element_type=jnp.float32)

  return pl.pallas_call(
      body,
      grid=(_GRID,),
      in_specs=[
          pl.BlockSpec((_ROWS_BLK, x.shape[1]), lambda i: (i, 0)),
          pl.BlockSpec(W.shape, lambda i: (0, 0)),
      ],
      out_specs=pl.BlockSpec((_ROWS_BLK, W.shape[1]), lambda i: (i, 0)),
      out_shape=jax.ShapeDtypeStruct((N, W.shape[1]), jnp.float32),
  )(x, W)


def _mm_agg(agg, b, W):
  D = agg.shape[2]

  def body(a_ref, b_ref, w_ref, o_ref):
    h = a_ref[0] + a_ref[1] + b_ref[...]
    o_ref[...] = jnp.dot(h, w_ref[...], preferred_element_type=jnp.float32)

  return pl.pallas_call(
      body,
      grid=(_GRID,),
      in_specs=[
          pl.BlockSpec((NC, _ROWS_BLK, D), lambda i: (0, i, 0)),
          pl.BlockSpec((1, D), lambda i: (0, 0)),
          pl.BlockSpec(W.shape, lambda i: (0, 0)),
      ],
      out_specs=pl.BlockSpec((_ROWS_BLK, W.shape[1]), lambda i: (i, 0)),
      out_shape=jax.ShapeDtypeStruct((N, W.shape[1]), jnp.float32),
  )(agg, b.reshape(1, D), W)


def _softmax_out(agg, b):
  D = agg.shape[2]

  def body(a_ref, b_ref, o_ref):
    z = a_ref[0] + a_ref[1] + b_ref[...]
    z = z - jnp.max(z, axis=1, keepdims=True)
    e = jnp.exp(z)
    o_ref[...] = e / jnp.sum(e, axis=1, keepdims=True)

  return pl.pallas_call(
      body,
      grid=(_GRID,),
      in_specs=[
          pl.BlockSpec((NC, _ROWS_BLK, D), lambda i: (0, i, 0)),
          pl.BlockSpec((1, D), lambda i: (0, 0)),
      ],
      out_specs=pl.BlockSpec((_ROWS_BLK, D), lambda i: (i, 0)),
      out_shape=jax.ShapeDtypeStruct((N, D), jnp.float32),
  )(agg, b.reshape(1, D))


def kernel(x, edge_index, W1, b1, W2, b2, W3, b3):
  src_p = edge_index[0].reshape(TOT_CHUNKS, CHUNK)
  dst_p = edge_index[1].reshape(TOT_CHUNKS, CHUNK)

  support1 = _mm_first(x, W1)
  agg1 = _spmm128(support1, src_p, dst_p)
  support2 = _mm_agg(agg1, b1, W2)
  agg2 = _spmm128(support2, src_p, dst_p)
  support3 = _mm_agg(agg2, b2, W3)
  agg3 = _spmm64(support3, src_p, dst_p)
  return _softmax_out(agg3, b3)


# speedup vs baseline: 3.2265x; 1.4996x over previous
import functools

import jax
import jax.numpy as jnp
from jax import lax
from jax.experimental import pallas as pl
from jax.experimental.pallas import tpu as pltpu
from jax.experimental.pallas import tpu_sc as plsc

N = 10000
E = 320000
NFEAT = 128
NHID = 128
NCLASS = 64

NC = 2
NS = 16
NW = NC * NS
CHUNK = 128
TOT_CHUNKS = E // CHUNK
KB = TOT_CHUNKS // NW
NA = TOT_CHUNKS - NW * KB
KA = KB + 1
G = 40
NACC = 10240
ROWS_PER_TILE = NACC // NS
COPIES_PER_TILE = ROWS_PER_TILE // CHUNK


def _make_spmm(D):
  mesh = plsc.VectorSubcoreMesh(core_axis_name="c", subcore_axis_name="s")

  @functools.partial(
      pl.kernel,
      out_type=jax.ShapeDtypeStruct((NC, NACC, D), jnp.float32),
      mesh=mesh,
      compiler_params=pltpu.CompilerParams(use_tc_tiling_on_sc=False),
      scratch_types=[
          pltpu.VMEM((G, CHUNK), jnp.int32),
          pltpu.VMEM((G, CHUNK), jnp.int32),
          pltpu.VMEM((CHUNK, D), jnp.float32),
          pltpu.VMEM((CHUNK, D), jnp.float32),
          pltpu.VMEM_SHARED((NACC, D), jnp.float32),
          pltpu.SemaphoreType.DMA,
          pltpu.SemaphoreType.DMA,
      ],
  )
  def spmm(support_hbm, src_hbm, dst_hbm, out_hbm, src_v, dst_v, rows0,
           rows1, acc, sem0, sem1):
    c = lax.axis_index("c")
    s = lax.axis_index("s")
    w = c * NS + s
    chunk0 = w * KB + jnp.minimum(w, NA)

    pltpu.async_copy(src_hbm.at[pl.ds(chunk0, G)], src_v, sem0)
    pltpu.async_copy(dst_hbm.at[pl.ds(chunk0, G)], dst_v, sem1)
    zeros16 = jnp.zeros((16,), jnp.float32)

    def zero_row(i, carry):
      for t in range(D // 16):
        rows0[i, pl.ds(t * 16, 16)] = zeros16
      return carry

    lax.fori_loop(0, CHUNK, zero_row, 0)
    base = s * ROWS_PER_TILE
    for t in range(COPIES_PER_TILE):
      pltpu.sync_copy(rows0, acc.at[pl.ds(base + t * CHUNK, CHUNK)])
    pltpu.make_async_copy(src_hbm.at[pl.ds(chunk0, G)], src_v, sem0).wait()
    pltpu.make_async_copy(dst_hbm.at[pl.ds(chunk0, G)], dst_v, sem1).wait()
    plsc.subcore_barrier()

    def run(kc):
      pltpu.async_copy(support_hbm.at[src_v.at[0]], rows0, sem0)
      for gi, glen in enumerate((G, kc - G)):
        pairs, rem = divmod(glen, 2)

        def body(i, carry):
          t0 = 2 * i
          pltpu.async_copy(support_hbm.at[src_v.at[t0 + 1]], rows1, sem1)
          pltpu.make_async_copy(support_hbm.at[src_v.at[t0]], rows0,
                                sem0).wait()
          pltpu.sync_copy(rows0, acc.at[dst_v.at[t0]], add=True)

          @pl.when(t0 + 2 < glen)
          def _():
            pltpu.async_copy(support_hbm.at[src_v.at[t0 + 2]], rows0, sem0)

          pltpu.make_async_copy(support_hbm.at[src_v.at[t0 + 1]], rows1,
                                sem1).wait()
          pltpu.sync_copy(rows1, acc.at[dst_v.at[t0 + 1]], add=True)
          return carry

        lax.fori_loop(0, pairs, body, 0)
        if rem:
          pltpu.make_async_copy(support_hbm.at[src_v.at[glen - 1]], rows0,
                                sem0).wait()
          pltpu.sync_copy(rows0, acc.at[dst_v.at[glen - 1]], add=True)
        if gi == 0:
          pltpu.sync_copy(src_hbm.at[pl.ds(chunk0 + G, kc - G)],
                          src_v.at[pl.ds(0, kc - G)])
          pltpu.sync_copy(dst_hbm.at[pl.ds(chunk0 + G, kc - G)],
                          dst_v.at[pl.ds(0, kc - G)])
          pltpu.async_copy(support_hbm.at[src_v.at[0]], rows0, sem0)

    @pl.when(w < NA)
    def _():
      run(KA)

    @pl.when(w >= NA)
    def _():
      run(KB)

    plsc.subcore_barrier()

    for t in range(COPIES_PER_TILE):
      r0 = base + t * CHUNK
      buf = rows0 if t % 2 == 0 else rows1
      pltpu.sync_copy(acc.at[pl.ds(r0, CHUNK)], buf)
      pltpu.sync_copy(buf, out_hbm.at[c, pl.ds(r0, CHUNK)])

  return spmm


_spmm128 = _make_spmm(NHID)
_spmm64 = _make_spmm(NCLASS)

_ROWS_BLK = 1000
_GRID = N // _ROWS_BLK


def _mm_first(x, W):
  def body(x_ref, w_ref, o_ref):
    o_ref[...] = jnp.dot(x_ref[...], w_ref[...],
                         preferred_element_type=jnp.float32)

  return pl.pallas_call(
      body,
      grid=(_GRID,),
      in_specs=[
          pl.BlockSpec((_ROWS_BLK, x.shape[1]), lambda i: (i, 0)),
          pl.BlockSpec(W.shape, lambda i: (0, 0)),
      ],
      out_specs=pl.BlockSpec((_ROWS_BLK, W.shape[1]), lambda i: (i, 0)),
      out_shape=jax.ShapeDtypeStruct((N, W.shape[1]), jnp.float32),
  )(x, W)


def _mm_agg(agg, b, W):
  D = agg.shape[2]

  def body(a_ref, b_ref, w_ref, o_ref):
    h = a_ref[0] + a_ref[1] + b_ref[...]
    o_ref[...] = jnp.dot(h, w_ref[...], preferred_element_type=jnp.float32)

  return pl.pallas_call(
      body,
      grid=(_GRID,),
      in_specs=[
          pl.BlockSpec((NC, _ROWS_BLK, D), lambda i: (0, i, 0)),
          pl.BlockSpec((1, D), lambda i: (0, 0)),
          pl.BlockSpec(W.shape, lambda i: (0, 0)),
      ],
      out_specs=pl.BlockSpec((_ROWS_BLK, W.shape[1]), lambda i: (i, 0)),
      out_shape=jax.ShapeDtypeStruct((N, W.shape[1]), jnp.float32),
  )(agg, b.reshape(1, D), W)


def _softmax_out(agg, b):
  D = agg.shape[2]

  def body(a_ref, b_ref, o_ref):
    z = a_ref[0] + a_ref[1] + b_ref[...]
    z = z - jnp.max(z, axis=1, keepdims=True)
    e = jnp.exp(z)
    o_ref[...] = e / jnp.sum(e, axis=1, keepdims=True)

  return pl.pallas_call(
      body,
      grid=(_GRID,),
      in_specs=[
          pl.BlockSpec((NC, _ROWS_BLK, D), lambda i: (0, i, 0)),
          pl.BlockSpec((1, D), lambda i: (0, 0)),
      ],
      out_specs=pl.BlockSpec((_ROWS_BLK, D), lambda i: (i, 0)),
      out_shape=jax.ShapeDtypeStruct((N, D), jnp.float32),
  )(agg, b.reshape(1, D))


def kernel(x, edge_index, W1, b1, W2, b2, W3, b3):
  src_p = edge_index[0].reshape(TOT_CHUNKS, CHUNK)
  dst_p = edge_index[1].reshape(TOT_CHUNKS, CHUNK)

  support1 = _mm_first(x, W1)
  agg1 = _spmm128(support1, src_p, dst_p)
  support2 = _mm_agg(agg1, b1, W2)
  agg2 = _spmm128(support2, src_p, dst_p)
  support3 = _mm_agg(agg2, b2, W3)
  agg3 = _spmm64(support3, src_p, dst_p)
  return _softmax_out(agg3, b3)


# speedup vs baseline: 3.3230x; 1.0299x over previous
import functools

import jax
import jax.numpy as jnp
from jax import lax
from jax.experimental import pallas as pl
from jax.experimental.pallas import tpu as pltpu
from jax.experimental.pallas import tpu_sc as plsc

N = 10000
E = 320000
NFEAT = 128
NHID = 128
NCLASS = 64

NC = 2
NS = 16
NW = NC * NS
CHUNK = 128
TOT_CHUNKS = E // CHUNK
KB = TOT_CHUNKS // NW
NA = TOT_CHUNKS - NW * KB
KA = KB + 1
G = 40
NACC = 10240
ROWS_PER_TILE = NACC // NS
COPIES_PER_TILE = ROWS_PER_TILE // CHUNK


def _make_spmm(D):
  mesh = plsc.VectorSubcoreMesh(core_axis_name="c", subcore_axis_name="s")

  @functools.partial(
      pl.kernel,
      out_type=jax.ShapeDtypeStruct((NC, NACC, D), jnp.float32),
      mesh=mesh,
      compiler_params=pltpu.CompilerParams(use_tc_tiling_on_sc=False),
      scratch_types=[
          pltpu.VMEM((G, CHUNK), jnp.int32),
          pltpu.VMEM((G, CHUNK), jnp.int32),
          pltpu.VMEM((CHUNK, D), jnp.float32),
          pltpu.VMEM((CHUNK, D), jnp.float32),
          pltpu.VMEM_SHARED((NACC, D), jnp.float32),
          pltpu.SemaphoreType.DMA,
          pltpu.SemaphoreType.DMA,
      ],
  )
  def spmm(support_hbm, ei_hbm, out_hbm, src_v, dst_v, rows0,
           rows1, acc, sem0, sem1):
    c = lax.axis_index("c")
    s = lax.axis_index("s")
    w = c * NS + s
    chunk0 = w * KB + jnp.minimum(w, NA)
    src_hbm = ei_hbm.at[0]
    dst_hbm = ei_hbm.at[1]

    pltpu.async_copy(src_hbm.at[pl.ds(chunk0, G)], src_v, sem0)
    pltpu.async_copy(dst_hbm.at[pl.ds(chunk0, G)], dst_v, sem1)
    zeros16 = jnp.zeros((16,), jnp.float32)

    def zero_row(i, carry):
      for t in range(D // 16):
        rows0[i, pl.ds(t * 16, 16)] = zeros16
      return carry

    lax.fori_loop(0, CHUNK, zero_row, 0)
    base = s * ROWS_PER_TILE
    for t in range(COPIES_PER_TILE):
      pltpu.sync_copy(rows0, acc.at[pl.ds(base + t * CHUNK, CHUNK)])
    pltpu.make_async_copy(src_hbm.at[pl.ds(chunk0, G)], src_v, sem0).wait()
    pltpu.make_async_copy(dst_hbm.at[pl.ds(chunk0, G)], dst_v, sem1).wait()
    plsc.subcore_barrier()

    def run(kc):
      pltpu.async_copy(support_hbm.at[src_v.at[0]], rows0, sem0)
      for gi, glen in enumerate((G, kc - G)):
        pairs, rem = divmod(glen, 2)

        def body(i, carry):
          t0 = 2 * i
          pltpu.async_copy(support_hbm.at[src_v.at[t0 + 1]], rows1, sem1)
          pltpu.make_async_copy(support_hbm.at[src_v.at[t0]], rows0,
                                sem0).wait()
          pltpu.sync_copy(rows0, acc.at[dst_v.at[t0]], add=True)

          @pl.when(t0 + 2 < glen)
          def _():
            pltpu.async_copy(support_hbm.at[src_v.at[t0 + 2]], rows0, sem0)

          pltpu.make_async_copy(support_hbm.at[src_v.at[t0 + 1]], rows1,
                                sem1).wait()
          pltpu.sync_copy(rows1, acc.at[dst_v.at[t0 + 1]], add=True)
          return carry

        lax.fori_loop(0, pairs, body, 0)
        if rem:
          pltpu.make_async_copy(support_hbm.at[src_v.at[glen - 1]], rows0,
                                sem0).wait()
          pltpu.sync_copy(rows0, acc.at[dst_v.at[glen - 1]], add=True)
        if gi == 0:
          pltpu.sync_copy(src_hbm.at[pl.ds(chunk0 + G, kc - G)],
                          src_v.at[pl.ds(0, kc - G)])
          pltpu.sync_copy(dst_hbm.at[pl.ds(chunk0 + G, kc - G)],
                          dst_v.at[pl.ds(0, kc - G)])
          pltpu.async_copy(support_hbm.at[src_v.at[0]], rows0, sem0)

    @pl.when(w < NA)
    def _():
      run(KA)

    @pl.when(w >= NA)
    def _():
      run(KB)

    plsc.subcore_barrier()

    for t in range(COPIES_PER_TILE):
      r0 = base + t * CHUNK
      buf = rows0 if t % 2 == 0 else rows1
      pltpu.sync_copy(acc.at[pl.ds(r0, CHUNK)], buf)
      pltpu.sync_copy(buf, out_hbm.at[c, pl.ds(r0, CHUNK)])

  return spmm


_spmm128 = _make_spmm(NHID)
_spmm64 = _make_spmm(NCLASS)

_ROWS_BLK = 1000
_GRID = N // _ROWS_BLK


def _mm_first(x, W):
  def body(x_ref, w_ref, o_ref):
    o_ref[...] = jnp.dot(x_ref[...], w_ref[...],
                         preferred_element_type=jnp.float32)

  return pl.pallas_call(
      body,
      grid=(_GRID,),
      in_specs=[
          pl.BlockSpec((_ROWS_BLK, x.shape[1]), lambda i: (i, 0)),
          pl.BlockSpec(W.shape, lambda i: (0, 0)),
      ],
      out_specs=pl.BlockSpec((_ROWS_BLK, W.shape[1]), lambda i: (i, 0)),
      out_shape=jax.ShapeDtypeStruct((N, W.shape[1]), jnp.float32),
  )(x, W)


def _mm_agg(agg, b, W):
  D = agg.shape[2]

  def body(a_ref, b_ref, w_ref, o_ref):
    h = a_ref[0] + a_ref[1] + b_ref[...]
    o_ref[...] = jnp.dot(h, w_ref[...], preferred_element_type=jnp.float32)

  return pl.pallas_call(
      body,
      grid=(_GRID,),
      in_specs=[
          pl.BlockSpec((NC, _ROWS_BLK, D), lambda i: (0, i, 0)),
          pl.BlockSpec((1, D), lambda i: (0, 0)),
          pl.BlockSpec(W.shape, lambda i: (0, 0)),
      ],
      out_specs=pl.BlockSpec((_ROWS_BLK, W.shape[1]), lambda i: (i, 0)),
      out_shape=jax.ShapeDtypeStruct((N, W.shape[1]), jnp.float32),
  )(agg, b.reshape(1, D), W)


def _softmax_out(agg, b):
  D = agg.shape[2]

  def body(a_ref, b_ref, o_ref):
    z = a_ref[0] + a_ref[1] + b_ref[...]
    z = z - jnp.max(z, axis=1, keepdims=True)
    e = jnp.exp(z)
    o_ref[...] = e / jnp.sum(e, axis=1, keepdims=True)

  return pl.pallas_call(
      body,
      grid=(_GRID,),
      in_specs=[
          pl.BlockSpec((NC, _ROWS_BLK, D), lambda i: (0, i, 0)),
          pl.BlockSpec((1, D), lambda i: (0, 0)),
      ],
      out_specs=pl.BlockSpec((_ROWS_BLK, D), lambda i: (i, 0)),
      out_shape=jax.ShapeDtypeStruct((N, D), jnp.float32),
  )(agg, b.reshape(1, D))


def kernel(x, edge_index, W1, b1, W2, b2, W3, b3):
  ei = edge_index.reshape(2, TOT_CHUNKS, CHUNK)

  support1 = _mm_first(x, W1)
  agg1 = _spmm128(support1, ei)
  support2 = _mm_agg(agg1, b1, W2)
  agg2 = _spmm128(support2, ei)
  support3 = _mm_agg(agg2, b2, W3)
  agg3 = _spmm64(support3, ei)
  return _softmax_out(agg3, b3)


# speedup vs baseline: 3.3636x; 1.0122x over previous
import functools

import jax
import jax.numpy as jnp
from jax import lax
from jax.experimental import pallas as pl
from jax.experimental.pallas import tpu as pltpu
from jax.experimental.pallas import tpu_sc as plsc

N = 10000
E = 320000
NFEAT = 128
NHID = 128
NCLASS = 64

NC = 2
NS = 16
NW = NC * NS
CHUNK = 128
TOT_CHUNKS = E // CHUNK
KB = TOT_CHUNKS // NW
NA = TOT_CHUNKS - NW * KB
KA = KB + 1
G = 40
NACC = 10240
ROWS_PER_TILE = NACC // NS
COPIES_PER_TILE = ROWS_PER_TILE // CHUNK


def _make_spmm(D):
  mesh = plsc.VectorSubcoreMesh(core_axis_name="c", subcore_axis_name="s")

  @functools.partial(
      pl.kernel,
      out_type=jax.ShapeDtypeStruct((NC, NACC, D), jnp.float32),
      mesh=mesh,
      compiler_params=pltpu.CompilerParams(use_tc_tiling_on_sc=False),
      scratch_types=[
          pltpu.VMEM((G, CHUNK), jnp.int32),
          pltpu.VMEM((G, CHUNK), jnp.int32),
          pltpu.VMEM((CHUNK, D), jnp.float32),
          pltpu.VMEM((CHUNK, D), jnp.float32),
          pltpu.VMEM_SHARED((NACC, D), jnp.float32),
          pltpu.SemaphoreType.DMA,
          pltpu.SemaphoreType.DMA,
      ],
  )
  def spmm(support_hbm, ei_hbm, out_hbm, src_v, dst_v, rows0,
           rows1, acc, sem0, sem1):
    c = lax.axis_index("c")
    s = lax.axis_index("s")
    w = c * NS + s
    chunk0 = w * KB + jnp.minimum(w, NA)
    src_hbm = ei_hbm.at[0]
    dst_hbm = ei_hbm.at[1]

    pltpu.async_copy(src_hbm.at[pl.ds(chunk0, G)], src_v, sem0)
    pltpu.async_copy(dst_hbm.at[pl.ds(chunk0, G)], dst_v, sem1)
    zeros16 = jnp.zeros((16,), jnp.float32)

    def zero_row(i, carry):
      for t in range(D // 16):
        rows0[i, pl.ds(t * 16, 16)] = zeros16
      return carry

    lax.fori_loop(0, CHUNK, zero_row, 0)
    base = s * ROWS_PER_TILE
    for t in range(COPIES_PER_TILE):
      pltpu.sync_copy(rows0, acc.at[pl.ds(base + t * CHUNK, CHUNK)])
    pltpu.make_async_copy(src_hbm.at[pl.ds(chunk0, G)], src_v, sem0).wait()
    pltpu.make_async_copy(dst_hbm.at[pl.ds(chunk0, G)], dst_v, sem1).wait()
    plsc.subcore_barrier()

    def run(kc):
      pltpu.async_copy(support_hbm.at[src_v.at[0]], rows0, sem0)
      for gi, glen in enumerate((G, kc - G)):
        pairs, rem = divmod(glen, 2)

        def body(i, carry):
          t0 = 2 * i
          pltpu.async_copy(support_hbm.at[src_v.at[t0 + 1]], rows1, sem1)
          pltpu.make_async_copy(support_hbm.at[src_v.at[t0]], rows0,
                                sem0).wait()
          pltpu.sync_copy(rows0, acc.at[dst_v.at[t0]], add=True)

          @pl.when(t0 + 2 < glen)
          def _():
            pltpu.async_copy(support_hbm.at[src_v.at[t0 + 2]], rows0, sem0)

          pltpu.make_async_copy(support_hbm.at[src_v.at[t0 + 1]], rows1,
                                sem1).wait()
          pltpu.sync_copy(rows1, acc.at[dst_v.at[t0 + 1]], add=True)
          return carry

        lax.fori_loop(0, pairs, body, 0)
        if rem:
          pltpu.make_async_copy(support_hbm.at[src_v.at[glen - 1]], rows0,
                                sem0).wait()
          pltpu.sync_copy(rows0, acc.at[dst_v.at[glen - 1]], add=True)
        if gi == 0:
          pltpu.sync_copy(src_hbm.at[pl.ds(chunk0 + G, kc - G)],
                          src_v.at[pl.ds(0, kc - G)])
          pltpu.sync_copy(dst_hbm.at[pl.ds(chunk0 + G, kc - G)],
                          dst_v.at[pl.ds(0, kc - G)])
          pltpu.async_copy(support_hbm.at[src_v.at[0]], rows0, sem0)

    @pl.when(w < NA)
    def _():
      run(KA)

    @pl.when(w >= NA)
    def _():
      run(KB)

    plsc.subcore_barrier()

    bufs = (rows0, rows1)
    sems = (sem0, sem1)

    def _slice(t):
      return pl.ds(base + t * CHUNK, CHUNK)

    pltpu.sync_copy(acc.at[_slice(0)], rows0)
    for t in range(COPIES_PER_TILE):
      pltpu.async_copy(bufs[t % 2], out_hbm.at[c, _slice(t)], sems[t % 2])
      if t + 1 < COPIES_PER_TILE:
        if t >= 1:
          pltpu.make_async_copy(bufs[(t + 1) % 2], out_hbm.at[c, _slice(t - 1)],
                                sems[(t + 1) % 2]).wait()
        pltpu.sync_copy(acc.at[_slice(t + 1)], bufs[(t + 1) % 2])
    for t in (COPIES_PER_TILE - 2, COPIES_PER_TILE - 1):
      pltpu.make_async_copy(bufs[t % 2], out_hbm.at[c, _slice(t)],
                            sems[t % 2]).wait()

  return spmm


_spmm128 = _make_spmm(NHID)
_spmm64 = _make_spmm(NCLASS)

_ROWS_BLK = 1000
_GRID = N // _ROWS_BLK


def _mm_first(x, W):
  def body(x_ref, w_ref, o_ref):
    o_ref[...] = jnp.dot(x_ref[...], w_ref[...],
                         preferred_element_type=jnp.float32)

  return pl.pallas_call(
      body,
      grid=(_GRID,),
      in_specs=[
          pl.BlockSpec((_ROWS_BLK, x.shape[1]), lambda i: (i, 0)),
          pl.BlockSpec(W.shape, lambda i: (0, 0)),
      ],
      out_specs=pl.BlockSpec((_ROWS_BLK, W.shape[1]), lambda i: (i, 0)),
      out_shape=jax.ShapeDtypeStruct((N, W.shape[1]), jnp.float32),
  )(x, W)


def _mm_agg(agg, b, W):
  D = agg.shape[2]

  def body(a_ref, b_ref, w_ref, o_ref):
    h = a_ref[0] + a_ref[1] + b_ref[...]
    o_ref[...] = jnp.dot(h, w_ref[...], preferred_element_type=jnp.float32)

  return pl.pallas_call(
      body,
      grid=(_GRID,),
      in_specs=[
          pl.BlockSpec((NC, _ROWS_BLK, D), lambda i: (0, i, 0)),
          pl.BlockSpec((1, D), lambda i: (0, 0)),
          pl.BlockSpec(W.shape, lambda i: (0, 0)),
      ],
      out_specs=pl.BlockSpec((_ROWS_BLK, W.shape[1]), lambda i: (i, 0)),
      out_shape=jax.ShapeDtypeStruct((N, W.shape[1]), jnp.float32),
  )(agg, b.reshape(1, D), W)


def _softmax_out(agg, b):
  D = agg.shape[2]

  def body(a_ref, b_ref, o_ref):
    z = a_ref[0] + a_ref[1] + b_ref[...]
    z = z - jnp.max(z, axis=1, keepdims=True)
    e = jnp.exp(z)
    o_ref[...] = e / jnp.sum(e, axis=1, keepdims=True)

  return pl.pallas_call(
      body,
      grid=(_GRID,),
      in_specs=[
          pl.BlockSpec((NC, _ROWS_BLK, D), lambda i: (0, i, 0)),
          pl.BlockSpec((1, D), lambda i: (0, 0)),
      ],
      out_specs=pl.BlockSpec((_ROWS_BLK, D), lambda i: (i, 0)),
      out_shape=jax.ShapeDtypeStruct((N, D), jnp.float32),
  )(agg, b.reshape(1, D))


def kernel(x, edge_index, W1, b1, W2, b2, W3, b3):
  ei = edge_index.reshape(2, TOT_CHUNKS, CHUNK)

  support1 = _mm_first(x, W1)
  agg1 = _spmm128(support1, ei)
  support2 = _mm_agg(agg1, b1, W2)
  agg2 = _spmm128(support2, ei)
  support3 = _mm_agg(agg2, b2, W3)
  agg3 = _spmm64(support3, ei)
  return _softmax_out(agg3, b3)
